# Initial kernel scaffold; baseline (speedup 1.0000x reference)
import jax
import jax.numpy as jnp
from jax.experimental import pallas as pl


def kernel(V_chanel, mu, W1, b1, W2, b2, W3, b3, W4, b4, W5, b5):
    raise NotImplementedError("write your pallas kernel here")


# speedup vs baseline: 2.5132x; 2.5132x over previous
import functools

import jax
import jax.numpy as jnp
from jax import lax
from jax.experimental import pallas as pl
from jax.experimental.pallas import tpu as pltpu
from jax.experimental.pallas import tpu_sc as plsc

_NBINS = 256
_MID = 64
_ITERS = 8
_NC, _NS, _L = 2, 16, 16
_NW = _NC * _NS
_B = 16
_H = 512
_W = 512
_HW = _H * _W
_HALF = _HW // 2
_CHUNK = 16384
_VPC = _CHUNK // _L
_NCH = _HALF // _CHUNK
_U = 8
_UH = 16

_mesh = plsc.VectorSubcoreMesh(
    core_axis_name="c", subcore_axis_name="s",
    num_cores=_NC, num_subcores=_NS)


def _minmax_body(v_hbm, mins_hbm, maxs_hbm, bufs, stage, sem0, sem1):
  c = lax.axis_index("c")
  s = lax.axis_index("s")
  wid = c * _NS + s
  base = wid * _HALF
  sems = (sem0, sem1)

  def src(k):
    return v_hbm.at[pl.ds(base + k * _CHUNK, _CHUNK)]

  mns = list(jnp.full((_L,), jnp.inf, jnp.float32) for _ in range(_U))
  mxs = list(jnp.full((_L,), -jnp.inf, jnp.float32) for _ in range(_U))

  pend = pltpu.async_copy(src(0), bufs.at[0], sems[0])
  for k in range(_NCH):
    nxt = None
    if k + 1 < _NCH:
      nxt = pltpu.async_copy(src(k + 1), bufs.at[(k + 1) % 2], sems[(k + 1) % 2])
    pend.wait()

    def step(i, carry2, _k=k):
      mns2, mxs2 = carry2
      off = i * _L
      new_mns, new_mxs = [], []
      for u in range(_U):
        x = bufs[_k % 2, pl.ds(off + u * _L, _L)]
        new_mns.append(jnp.minimum(mns2[u], x))
        new_mxs.append(jnp.maximum(mxs2[u], x))
      return tuple(new_mns), tuple(new_mxs)

    mns, mxs = plsc.parallel_loop(
        0, _VPC, step=_U, carry=(tuple(mns), tuple(mxs)))(step)
    pend = nxt
  stage[pl.ds(0, _L)] = functools.reduce(jnp.minimum, mns)
  stage[pl.ds(_L, _L)] = functools.reduce(jnp.maximum, mxs)
  pltpu.sync_copy(stage.at[pl.ds(0, _L)], mins_hbm.at[pl.ds(wid * _L, _L)])
  pltpu.sync_copy(stage.at[pl.ds(_L, _L)], maxs_hbm.at[pl.ds(wid * _L, _L)])


_minmax_call = pl.kernel(
    _minmax_body,
    out_type=(jax.ShapeDtypeStruct((_NW * _L,), jnp.float32),
              jax.ShapeDtypeStruct((_NW * _L,), jnp.float32)),
    mesh=_mesh,
    scratch_types=[pltpu.VMEM((2, _CHUNK), jnp.float32),
                   pltpu.VMEM((2 * _L,), jnp.float32),
                   pltpu.SemaphoreType.DMA,
                   pltpu.SemaphoreType.DMA],
    compiler_params=pltpu.CompilerParams(needs_layout_passes=False),
)


def _hist_body(v_hbm, mins_hbm, maxs_hbm, hist_hbm, bufs, hvals, stage,
               sem0, sem1):
  c = lax.axis_index("c")
  s = lax.axis_index("s")
  wid = c * _NS + s
  b = wid // 2
  base = wid * _HALF
  sems = (sem0, sem1)

  def src(k):
    return v_hbm.at[pl.ds(base + k * _CHUNK, _CHUNK)]

  pltpu.sync_copy(mins_hbm.at[pl.ds(b * 2 * _L, 2 * _L)], stage)
  mnv = jnp.minimum(stage[pl.ds(0, _L)], stage[pl.ds(_L, _L)])
  pltpu.sync_copy(maxs_hbm.at[pl.ds(b * 2 * _L, 2 * _L)], stage)
  mxv = jnp.maximum(stage[pl.ds(0, _L)], stage[pl.ds(_L, _L)])
  mn_s = mnv[0]
  mx_s = mxv[0]
  for i in range(1, _L):
    mn_s = jnp.minimum(mn_s, mnv[i])
    mx_s = jnp.maximum(mx_s, mxv[i])
  mn = jnp.broadcast_to(mn_s, (_L,))
  mx = jnp.broadcast_to(mx_s, (_L,))
  rng = mx - mn
  safe = jnp.where(rng == 0.0, jnp.float32(1.0), rng)
  inv = jnp.float32(_NBINS) / safe

  zero = jnp.zeros((_L,), jnp.float32)
  for j in range(_NBINS // _L):
    hvals[pl.ds(j * _L, _L)] = zero

  ones = jnp.ones((_L,), jnp.float32)

  pend = pltpu.async_copy(src(0), bufs.at[0], sems[0])
  for k in range(_NCH):
    nxt = None
    if k + 1 < _NCH:
      nxt = pltpu.async_copy(src(k + 1), bufs.at[(k + 1) % 2], sems[(k + 1) % 2])
    pend.wait()

    def step(i, _k=k):
      off = i * _L
      for u in range(_UH):
        x = bufs[_k % 2, pl.ds(off + u * _L, _L)]
        t = (x - mn) * inv
        idx = jnp.minimum(t, jnp.float32(_NBINS - 1)).astype(jnp.int32)
        plsc.addupdate_scatter(hvals, [idx], ones)

    plsc.parallel_loop(0, _VPC, step=_UH)(step)
    pend = nxt

  pltpu.sync_copy(hvals, hist_hbm.at[pl.ds(wid * _NBINS, _NBINS)])


_hist_call = pl.kernel(
    _hist_body,
    out_type=jax.ShapeDtypeStruct((_NW * _NBINS,), jnp.float32),
    mesh=_mesh,
    scratch_types=[pltpu.VMEM((2, _CHUNK), jnp.float32),
                   pltpu.VMEM((_NBINS,), jnp.float32),
                   pltpu.VMEM((2 * _L,), jnp.float32),
                   pltpu.SemaphoreType.DMA,
                   pltpu.SemaphoreType.DMA],
    compiler_params=pltpu.CompilerParams(needs_layout_passes=False),
)


def _lrelu(x):
  return jnp.where(x >= 0, x, 0.01 * x)


def _mlp_curve_body(he_ref, ho_ref, mne_ref, mno_ref, mxe_ref, mxo_ref, mu_ref,
                    w1, b1, w2, b2, w3, b3, w4, b4, w5, b5, v_ref, o_ref,
                    a_scr):
  b = pl.program_id(0)

  @pl.when(b == 0)
  def _():
    counts = he_ref[...] + ho_ref[...]
    h = counts * jnp.float32(1.0 / _HW)
    mn = jnp.min(jnp.minimum(mne_ref[...], mno_ref[...]), axis=1,
                 keepdims=True)
    mx = jnp.max(jnp.maximum(mxe_ref[...], mxo_ref[...]), axis=1,
                 keepdims=True)
    vec = jnp.concatenate([h, mn, mx, mu_ref[...]], axis=1)
    x = _lrelu(vec @ w1[...] + b1[...])
    x = _lrelu(x @ w2[...] + b2[...])
    x = _lrelu(jnp.concatenate([x, vec], axis=1) @ w3[...] + b3[...])
    x = _lrelu(x @ w4[...] + b4[...])
    a_scr[...] = _lrelu(x @ w5[...] + b5[...])

  sel = (lax.broadcasted_iota(jnp.int32, (_B, 1), 0) == b).astype(jnp.float32)
  arow = jnp.sum(a_scr[...] * sel, axis=0, keepdims=True)
  x = v_ref[0]
  for i in range(_ITERS):
    a = arow[:, i:i + 1]
    x = x * ((1.0 + a) - a * x)
  o_ref[0] = x


def kernel(V_chanel, mu, W1, b1, W2, b2, W3, b3, W4, b4, W5, b5):
  v_flat = V_chanel.reshape(_B * _HW)
  mins, maxs = _minmax_call(v_flat)
  hist = _hist_call(v_flat, mins, maxs)

  hist32 = hist.reshape(_NW, _NBINS)
  mins32 = mins.reshape(_NW, _L)
  maxs32 = maxs.reshape(_NW, _L)

  v3 = V_chanel.reshape(_B, _H, _W)
  small = lambda shape: pl.BlockSpec(shape, lambda b: tuple(0 for _ in shape))
  out = pl.pallas_call(
      _mlp_curve_body,
      grid=(_B,),
      in_specs=[
          small((_B, _NBINS)), small((_B, _NBINS)),
          small((_B, _L)), small((_B, _L)),
          small((_B, _L)), small((_B, _L)),
          small((_B, 1)),
          small((_NBINS + 3, _MID)), small((_MID,)),
          small((_MID, _MID)), small((_MID,)),
          small((_MID + _NBINS + 3, _MID)), small((_MID,)),
          small((_MID, _MID)), small((_MID,)),
          small((_MID, _ITERS)), small((_ITERS,)),
          pl.BlockSpec((1, _H, _W), lambda b: (b, 0, 0)),
      ],
      out_specs=pl.BlockSpec((1, _H, _W), lambda b: (b, 0, 0)),
      out_shape=jax.ShapeDtypeStruct((_B, _H, _W), jnp.float32),
      scratch_shapes=[pltpu.VMEM((_B, _ITERS), jnp.float32)],
  )(hist32[0::2], hist32[1::2], mins32[0::2], mins32[1::2],
    maxs32[0::2], maxs32[1::2], mu,
    W1, b1, W2, b2, W3, b3, W4, b4, W5, b5, v3)
  return out.reshape(V_chanel.shape)


# speedup vs baseline: 2.6101x; 1.0385x over previous
import functools

import jax
import jax.numpy as jnp
from jax import lax
from jax.experimental import pallas as pl
from jax.experimental.pallas import tpu as pltpu
from jax.experimental.pallas import tpu_sc as plsc

_NBINS = 256
_MID = 64
_ITERS = 8
_NC, _NS, _L = 2, 16, 16
_NW = _NC * _NS
_B = 16
_H = 512
_W = 512
_HW = _H * _W
_HALF = _HW // 2
_CHUNK = 16384
_VPC = _CHUNK // _L
_NCH = _HALF // _CHUNK
_U = 8
_UH = 16

_mesh = plsc.VectorSubcoreMesh(
    core_axis_name="c", subcore_axis_name="s",
    num_cores=_NC, num_subcores=_NS)


def _sc_body(v_hbm, mins_hbm, maxs_hbm, hist_hbm, bufs, hvals, stage, stage2,
             shared, sem0, sem1):
  c = lax.axis_index("c")
  s = lax.axis_index("s")
  wid = c * _NS + s
  base = wid * _HALF
  sems = (sem0, sem1)

  def src(k):
    return v_hbm.at[pl.ds(base + k * _CHUNK, _CHUNK)]

  mns = list(jnp.full((_L,), jnp.inf, jnp.float32) for _ in range(_U))
  mxs = list(jnp.full((_L,), -jnp.inf, jnp.float32) for _ in range(_U))
  pend = pltpu.async_copy(src(0), bufs.at[0], sems[0])
  for k in range(_NCH):
    nxt = None
    if k + 1 < _NCH:
      nxt = pltpu.async_copy(src(k + 1), bufs.at[(k + 1) % 2],
                             sems[(k + 1) % 2])
    pend.wait()

    def step(i, carry2, _k=k):
      mns2, mxs2 = carry2
      off = i * _L
      new_mns, new_mxs = [], []
      for u in range(_U):
        x = bufs[_k % 2, pl.ds(off + u * _L, _L)]
        new_mns.append(jnp.minimum(mns2[u], x))
        new_mxs.append(jnp.maximum(mxs2[u], x))
      return tuple(new_mns), tuple(new_mxs)

    mns, mxs = plsc.parallel_loop(
        0, _VPC, step=_U, carry=(tuple(mns), tuple(mxs)))(step)
    pend = nxt
  own_mn = functools.reduce(jnp.minimum, mns)
  own_mx = functools.reduce(jnp.maximum, mxs)
  stage[pl.ds(0, _L)] = own_mn
  stage[pl.ds(_L, _L)] = own_mx
  pltpu.sync_copy(stage.at[pl.ds(0, _L)], mins_hbm.at[pl.ds(wid * _L, _L)])
  pltpu.sync_copy(stage.at[pl.ds(_L, _L)], maxs_hbm.at[pl.ds(wid * _L, _L)])
  pltpu.sync_copy(stage, shared.at[s])
  plsc.subcore_barrier()
  pltpu.sync_copy(shared.at[s ^ 1], stage2)
  mnv = jnp.minimum(own_mn, stage2[pl.ds(0, _L)])
  mxv = jnp.maximum(own_mx, stage2[pl.ds(_L, _L)])
  mn_s = mnv[0]
  mx_s = mxv[0]
  for i in range(1, _L):
    mn_s = jnp.minimum(mn_s, mnv[i])
    mx_s = jnp.maximum(mx_s, mxv[i])
  mn = jnp.broadcast_to(mn_s, (_L,))
  mx = jnp.broadcast_to(mx_s, (_L,))
  rng = mx - mn
  safe = jnp.where(rng == 0.0, jnp.float32(1.0), rng)
  inv = jnp.float32(_NBINS) / safe

  zero = jnp.zeros((_L,), jnp.float32)
  for j in range(_NBINS // _L):
    hvals[pl.ds(j * _L, _L)] = zero

  ones = jnp.ones((_L,), jnp.float32)
  pend = pltpu.async_copy(src(0), bufs.at[0], sems[0])
  for k in range(_NCH):
    nxt = None
    if k + 1 < _NCH:
      nxt = pltpu.async_copy(src(k + 1), bufs.at[(k + 1) % 2],
                             sems[(k + 1) % 2])
    pend.wait()

    def step(i, _k=k):
      off = i * _L
      for u in range(_UH):
        x = bufs[_k % 2, pl.ds(off + u * _L, _L)]
        t = (x - mn) * inv
        idx = jnp.minimum(t, jnp.float32(_NBINS - 1)).astype(jnp.int32)
        plsc.addupdate_scatter(hvals, [idx], ones)

    plsc.parallel_loop(0, _VPC, step=_UH)(step)
    pend = nxt

  pltpu.sync_copy(hvals, hist_hbm.at[pl.ds(wid * _NBINS, _NBINS)])


_sc_call = pl.kernel(
    _sc_body,
    out_type=(jax.ShapeDtypeStruct((_NW * _L,), jnp.float32),
              jax.ShapeDtypeStruct((_NW * _L,), jnp.float32),
              jax.ShapeDtypeStruct((_NW * _NBINS,), jnp.float32)),
    mesh=_mesh,
    scratch_types=[pltpu.VMEM((2, _CHUNK), jnp.float32),
                   pltpu.VMEM((_NBINS,), jnp.float32),
                   pltpu.VMEM((2 * _L,), jnp.float32),
                   pltpu.VMEM((2 * _L,), jnp.float32),
                   pltpu.VMEM_SHARED((_NS, 2 * _L), jnp.float32),
                   pltpu.SemaphoreType.DMA,
                   pltpu.SemaphoreType.DMA],
    compiler_params=pltpu.CompilerParams(needs_layout_passes=False),
)


def _lrelu(x):
  return jnp.where(x >= 0, x, 0.01 * x)


def _mlp_body(hist_ref, mins_ref, maxs_ref, mu_ref,
              w1, b1, w2, b2, w3, b3, w4, b4, w5, b5, out_ref):
  h3 = hist_ref[...].reshape(_B, 2, _NBINS)
  counts = h3[:, 0, :] + h3[:, 1, :]
  h = counts * jnp.float32(1.0 / _HW)
  m3 = mins_ref[...].reshape(_B, 2, _L)
  x3 = maxs_ref[...].reshape(_B, 2, _L)
  mn = jnp.min(jnp.minimum(m3[:, 0, :], m3[:, 1, :]), axis=1, keepdims=True)
  mx = jnp.max(jnp.maximum(x3[:, 0, :], x3[:, 1, :]), axis=1, keepdims=True)
  vec = jnp.concatenate([h, mn, mx, mu_ref[...]], axis=1)
  x = _lrelu(vec @ w1[...] + b1[...])
  x = _lrelu(x @ w2[...] + b2[...])
  x = _lrelu(jnp.concatenate([x, vec], axis=1) @ w3[...] + b3[...])
  x = _lrelu(x @ w4[...] + b4[...])
  out_ref[...] = _lrelu(x @ w5[...] + b5[...])


def _curve_body(a_ref, v_ref, o_ref):
  b = pl.program_id(0)
  x = v_ref[0]
  for i in range(_ITERS):
    a = a_ref[b, i]
    x = x * ((1.0 + a) - a * x)
  o_ref[0] = x


def kernel(V_chanel, mu, W1, b1, W2, b2, W3, b3, W4, b4, W5, b5):
  v_flat = V_chanel.reshape(_B * _HW)
  mins, maxs, hist = _sc_call(v_flat)

  alphas = pl.pallas_call(
      _mlp_body,
      out_shape=jax.ShapeDtypeStruct((_B, _ITERS), jnp.float32),
  )(hist.reshape(_NW, _NBINS), mins.reshape(_NW, _L), maxs.reshape(_NW, _L),
    mu, W1, b1, W2, b2, W3, b3, W4, b4, W5, b5)

  v3 = V_chanel.reshape(_B, _H, _W)
  out = pl.pallas_call(
      _curve_body,
      grid=(_B,),
      in_specs=[
          pl.BlockSpec((_B, _ITERS), lambda b: (0, 0),
                       memory_space=pltpu.SMEM),
          pl.BlockSpec((1, _H, _W), lambda b: (b, 0, 0)),
      ],
      out_specs=pl.BlockSpec((1, _H, _W), lambda b: (b, 0, 0)),
      out_shape=jax.ShapeDtypeStruct((_B, _H, _W), jnp.float32),
  )(alphas, v3)
  return out.reshape(V_chanel.shape)


# speedup vs baseline: 3.4683x; 1.3288x over previous
import functools

import jax
import jax.numpy as jnp
from jax import lax
from jax.experimental import pallas as pl
from jax.experimental.pallas import tpu as pltpu
from jax.experimental.pallas import tpu_sc as plsc

_NBINS = 256
_MID = 64
_ITERS = 8
_NC, _NS, _L = 2, 16, 16
_NW = _NC * _NS
_B = 16
_H = 512
_W = 512
_HW = _H * _W
_HALF = _HW // 2
_CHR = 64
_NCH = (_H // 2) // _CHR
_U = 8
_UH = 16

_mesh = plsc.VectorSubcoreMesh(
    core_axis_name="c", subcore_axis_name="s",
    num_cores=_NC, num_subcores=_NS)


def _sc_body(v_hbm, mins_hbm, maxs_hbm, hist_hbm, bufs, hvals, stage, stage2,
             shared, sem0, sem1):
  c = lax.axis_index("c")
  s = lax.axis_index("s")
  wid = c * _NS + s
  b = wid // 2
  row0 = (wid % 2) * (_H // 2)
  sems = (sem0, sem1)

  def src(k):
    return v_hbm.at[b, pl.ds(row0 + k * _CHR, _CHR), :]

  mns = list(jnp.full((_L,), jnp.inf, jnp.float32) for _ in range(_U))
  mxs = list(jnp.full((_L,), -jnp.inf, jnp.float32) for _ in range(_U))
  pend = pltpu.async_copy(src(0), bufs.at[0], sems[0])
  for k in range(_NCH):
    nxt = None
    if k + 1 < _NCH:
      nxt = pltpu.async_copy(src(k + 1), bufs.at[(k + 1) % 2],
                             sems[(k + 1) % 2])
    pend.wait()

    def step(i, carry2, _k=k):
      mns2, mxs2 = carry2
      new_mns, new_mxs = list(mns2), list(mxs2)
      row = i >> 1
      col = (i & 1) * (_W // 2)
      for u in range(_W // (2 * _L)):
        x = bufs[_k % 2, row, pl.ds(col + u * _L, _L)]
        new_mns[u % _U] = jnp.minimum(new_mns[u % _U], x)
        new_mxs[u % _U] = jnp.maximum(new_mxs[u % _U], x)
      return tuple(new_mns), tuple(new_mxs)

    mns, mxs = plsc.parallel_loop(
        0, 2 * _CHR, carry=(tuple(mns), tuple(mxs)))(step)
    pend = nxt
  own_mn = functools.reduce(jnp.minimum, mns)
  own_mx = functools.reduce(jnp.maximum, mxs)
  stage[pl.ds(0, _L)] = own_mn
  stage[pl.ds(_L, _L)] = own_mx
  pltpu.sync_copy(stage.at[pl.ds(0, _L)], mins_hbm.at[pl.ds(wid * _L, _L)])
  pltpu.sync_copy(stage.at[pl.ds(_L, _L)], maxs_hbm.at[pl.ds(wid * _L, _L)])
  pltpu.sync_copy(stage, shared.at[s])
  plsc.subcore_barrier()
  pltpu.sync_copy(shared.at[s ^ 1], stage2)
  mnv = jnp.minimum(own_mn, stage2[pl.ds(0, _L)])
  mxv = jnp.maximum(own_mx, stage2[pl.ds(_L, _L)])
  mn_s = mnv[0]
  mx_s = mxv[0]
  for i in range(1, _L):
    mn_s = jnp.minimum(mn_s, mnv[i])
    mx_s = jnp.maximum(mx_s, mxv[i])
  mn = jnp.broadcast_to(mn_s, (_L,))
  mx = jnp.broadcast_to(mx_s, (_L,))
  rng = mx - mn
  safe = jnp.where(rng == 0.0, jnp.float32(1.0), rng)
  inv = jnp.float32(_NBINS) / safe

  zero = jnp.zeros((_L,), jnp.float32)
  for j in range(_NBINS // _L):
    hvals[pl.ds(j * _L, _L)] = zero

  ones = jnp.ones((_L,), jnp.float32)
  pend = pltpu.async_copy(src(0), bufs.at[0], sems[0])
  for k in range(_NCH):
    nxt = None
    if k + 1 < _NCH:
      nxt = pltpu.async_copy(src(k + 1), bufs.at[(k + 1) % 2],
                             sems[(k + 1) % 2])
    pend.wait()

    def step(i, _k=k):
      row = i >> 1
      col = (i & 1) * (_W // 2)
      for u in range(_W // (2 * _L)):
        x = bufs[_k % 2, row, pl.ds(col + u * _L, _L)]
        t = (x - mn) * inv
        idx = jnp.minimum(t, jnp.float32(_NBINS - 1)).astype(jnp.int32)
        plsc.addupdate_scatter(hvals, [idx], ones)

    plsc.parallel_loop(0, 2 * _CHR)(step)
    pend = nxt

  pltpu.sync_copy(hvals, hist_hbm.at[pl.ds(wid * _NBINS, _NBINS)])


_sc_call = pl.kernel(
    _sc_body,
    out_type=(jax.ShapeDtypeStruct((_NW * _L,), jnp.float32),
              jax.ShapeDtypeStruct((_NW * _L,), jnp.float32),
              jax.ShapeDtypeStruct((_NW * _NBINS,), jnp.float32)),
    mesh=_mesh,
    scratch_types=[pltpu.VMEM((2, _CHR, _W), jnp.float32),
                   pltpu.VMEM((_NBINS,), jnp.float32),
                   pltpu.VMEM((2 * _L,), jnp.float32),
                   pltpu.VMEM((2 * _L,), jnp.float32),
                   pltpu.VMEM_SHARED((_NS, 2 * _L), jnp.float32),
                   pltpu.SemaphoreType.DMA,
                   pltpu.SemaphoreType.DMA],
    compiler_params=pltpu.CompilerParams(needs_layout_passes=False,
                                         use_tc_tiling_on_sc=True),
)


def _lrelu(x):
  return jnp.where(x >= 0, x, 0.01 * x)


def _mlp_body(hist_ref, mins_ref, maxs_ref, mu_ref,
              w1, b1, w2, b2, w3, b3, w4, b4, w5, b5, out_ref):
  h3 = hist_ref[...].reshape(_B, 2, _NBINS)
  counts = h3[:, 0, :] + h3[:, 1, :]
  h = counts * jnp.float32(1.0 / _HW)
  m3 = mins_ref[...].reshape(_B, 2, _L)
  x3 = maxs_ref[...].reshape(_B, 2, _L)
  mn = jnp.min(jnp.minimum(m3[:, 0, :], m3[:, 1, :]), axis=1, keepdims=True)
  mx = jnp.max(jnp.maximum(x3[:, 0, :], x3[:, 1, :]), axis=1, keepdims=True)
  vec = jnp.concatenate([h, mn, mx, mu_ref[...]], axis=1)
  x = _lrelu(vec @ w1[...] + b1[...])
  x = _lrelu(x @ w2[...] + b2[...])
  x = _lrelu(jnp.concatenate([x, vec], axis=1) @ w3[...] + b3[...])
  x = _lrelu(x @ w4[...] + b4[...])
  out_ref[...] = _lrelu(x @ w5[...] + b5[...])


def _curve_body(a_ref, v_ref, o_ref):
  b = pl.program_id(0)
  x = v_ref[0]
  for i in range(_ITERS):
    a = a_ref[b, i]
    x = x * ((1.0 + a) - a * x)
  o_ref[0] = x


def kernel(V_chanel, mu, W1, b1, W2, b2, W3, b3, W4, b4, W5, b5):
  v3 = V_chanel.reshape(_B, _H, _W)
  mins, maxs, hist = _sc_call(v3)

  alphas = pl.pallas_call(
      _mlp_body,
      out_shape=jax.ShapeDtypeStruct((_B, _ITERS), jnp.float32),
  )(hist.reshape(_NW, _NBINS), mins.reshape(_NW, _L), maxs.reshape(_NW, _L),
    mu, W1, b1, W2, b2, W3, b3, W4, b4, W5, b5)

  out = pl.pallas_call(
      _curve_body,
      grid=(_B,),
      in_specs=[
          pl.BlockSpec((_B, _ITERS), lambda b: (0, 0),
                       memory_space=pltpu.SMEM),
          pl.BlockSpec((1, _H, _W), lambda b: (b, 0, 0)),
      ],
      out_specs=pl.BlockSpec((1, _H, _W), lambda b: (b, 0, 0)),
      out_shape=jax.ShapeDtypeStruct((_B, _H, _W), jnp.float32),
  )(alphas, v3)
  return out.reshape(V_chanel.shape)


# speedup vs baseline: 3.6181x; 1.0432x over previous
import functools

import jax
import jax.numpy as jnp
from jax import lax
from jax.experimental import pallas as pl
from jax.experimental.pallas import tpu as pltpu
from jax.experimental.pallas import tpu_sc as plsc

_NBINS = 256
_MID = 64
_ITERS = 8
_NC, _NS, _L = 2, 16, 16
_NW = _NC * _NS
_B = 16
_H = 512
_W = 512
_HW = _H * _W
_HALF = _HW // 2
_CHR = 64
_NCH = (_H // 2) // _CHR
_U = 8
_UH = 16

_mesh = plsc.VectorSubcoreMesh(
    core_axis_name="c", subcore_axis_name="s",
    num_cores=_NC, num_subcores=_NS)


def _sc_body(v_hbm, mins_hbm, maxs_hbm, hist_hbm, bufs, hvals, stage, stage2,
             shared, sem0, sem1):
  c = lax.axis_index("c")
  s = lax.axis_index("s")
  wid = c * _NS + s
  b = wid // 2
  row0 = (wid % 2) * (_H // 2)
  sems = (sem0, sem1)

  def src(k):
    return v_hbm.at[b, pl.ds(row0 + k * _CHR, _CHR), :]

  mns = list(jnp.full((_L,), jnp.inf, jnp.float32) for _ in range(_U))
  mxs = list(jnp.full((_L,), -jnp.inf, jnp.float32) for _ in range(_U))
  pend = pltpu.async_copy(src(0), bufs.at[0], sems[0])
  for k in range(_NCH):
    nxt = None
    if k + 1 < _NCH:
      nxt = pltpu.async_copy(src(k + 1), bufs.at[(k + 1) % 2],
                             sems[(k + 1) % 2])
    pend.wait()

    def step(i, carry2, _k=k):
      mns2, mxs2 = carry2
      new_mns, new_mxs = list(mns2), list(mxs2)
      row = i >> 1
      col = (i & 1) * (_W // 2)
      for u in range(_W // (2 * _L)):
        x = bufs[_k % 2, row, pl.ds(col + u * _L, _L)]
        new_mns[u % _U] = jnp.minimum(new_mns[u % _U], x)
        new_mxs[u % _U] = jnp.maximum(new_mxs[u % _U], x)
      return tuple(new_mns), tuple(new_mxs)

    mns, mxs = plsc.parallel_loop(
        0, 2 * _CHR, carry=(tuple(mns), tuple(mxs)))(step)
    pend = nxt
  own_mn = functools.reduce(jnp.minimum, mns)
  own_mx = functools.reduce(jnp.maximum, mxs)
  stage[pl.ds(0, _L)] = own_mn
  stage[pl.ds(_L, _L)] = own_mx
  pltpu.sync_copy(stage.at[pl.ds(0, _L)], mins_hbm.at[pl.ds(wid * _L, _L)])
  pltpu.sync_copy(stage.at[pl.ds(_L, _L)], maxs_hbm.at[pl.ds(wid * _L, _L)])
  pltpu.sync_copy(stage, shared.at[s])
  plsc.subcore_barrier()
  pltpu.sync_copy(shared.at[s ^ 1], stage2)
  mnv = jnp.minimum(own_mn, stage2[pl.ds(0, _L)])
  mxv = jnp.maximum(own_mx, stage2[pl.ds(_L, _L)])
  mn_s = mnv[0]
  mx_s = mxv[0]
  for i in range(1, _L):
    mn_s = jnp.minimum(mn_s, mnv[i])
    mx_s = jnp.maximum(mx_s, mxv[i])
  mn = jnp.broadcast_to(mn_s, (_L,))
  mx = jnp.broadcast_to(mx_s, (_L,))
  rng = mx - mn
  safe = jnp.where(rng == 0.0, jnp.float32(1.0), rng)
  inv = jnp.float32(_NBINS) / safe

  zero = jnp.zeros((_L,), jnp.float32)
  for j in range(_NBINS // _L):
    hvals[pl.ds(j * _L, _L)] = zero

  ones = jnp.ones((_L,), jnp.float32)
  pend = pltpu.async_copy(src(0), bufs.at[0], sems[0])
  for k in range(_NCH):
    nxt = None
    if k + 1 < _NCH:
      nxt = pltpu.async_copy(src(k + 1), bufs.at[(k + 1) % 2],
                             sems[(k + 1) % 2])
    pend.wait()

    def step(i, _k=k):
      row = i >> 1
      col = (i & 1) * (_W // 2)
      for u in range(_W // (2 * _L)):
        x = bufs[_k % 2, row, pl.ds(col + u * _L, _L)]
        t = (x - mn) * inv
        idx = jnp.minimum(t, jnp.float32(_NBINS - 1)).astype(jnp.int32)
        plsc.addupdate_scatter(hvals, [idx], ones)

    plsc.parallel_loop(0, 2 * _CHR)(step)
    pend = nxt

  pltpu.sync_copy(hvals, hist_hbm.at[pl.ds(wid * _NBINS, _NBINS)])


_sc_call = pl.kernel(
    _sc_body,
    out_type=(jax.ShapeDtypeStruct((_NW * _L,), jnp.float32),
              jax.ShapeDtypeStruct((_NW * _L,), jnp.float32),
              jax.ShapeDtypeStruct((_NW * _NBINS,), jnp.float32)),
    mesh=_mesh,
    scratch_types=[pltpu.VMEM((2, _CHR, _W), jnp.float32),
                   pltpu.VMEM((_NBINS,), jnp.float32),
                   pltpu.VMEM((2 * _L,), jnp.float32),
                   pltpu.VMEM((2 * _L,), jnp.float32),
                   pltpu.VMEM_SHARED((_NS, 2 * _L), jnp.float32),
                   pltpu.SemaphoreType.DMA,
                   pltpu.SemaphoreType.DMA],
    compiler_params=pltpu.CompilerParams(needs_layout_passes=False,
                                         use_tc_tiling_on_sc=True),
)


def _lrelu(x):
  return jnp.where(x >= 0, x, 0.01 * x)


def _mlp_body(hist_ref, mins_ref, maxs_ref, mu_ref,
              w1, b1, w2, b2, w3, b3, w4, b4, w5, b5, out_ref):
  h3 = hist_ref[...].reshape(_B, 2, _NBINS)
  counts = h3[:, 0, :] + h3[:, 1, :]
  h = counts * jnp.float32(1.0 / _HW)
  m3 = mins_ref[...].reshape(_B, 2, _L)
  x3 = maxs_ref[...].reshape(_B, 2, _L)
  mn = jnp.min(jnp.minimum(m3[:, 0, :], m3[:, 1, :]), axis=1, keepdims=True)
  mx = jnp.max(jnp.maximum(x3[:, 0, :], x3[:, 1, :]), axis=1, keepdims=True)
  vec = jnp.concatenate([h, mn, mx, mu_ref[...]], axis=1)
  x = _lrelu(vec @ w1[...] + b1[...])
  x = _lrelu(x @ w2[...] + b2[...])
  x = _lrelu(jnp.concatenate([x, vec], axis=1) @ w3[...] + b3[...])
  x = _lrelu(x @ w4[...] + b4[...])
  out_ref[...] = _lrelu(x @ w5[...] + b5[...])


_CB = 2


def _curve_body(a_ref, v_ref, o_ref):
  g = pl.program_id(0)
  for j in range(_CB):
    x = v_ref[j]
    for i in range(_ITERS):
      a = a_ref[g * _CB + j, i]
      x = x * ((1.0 + a) - a * x)
    o_ref[j] = x


def kernel(V_chanel, mu, W1, b1, W2, b2, W3, b3, W4, b4, W5, b5):
  v3 = V_chanel.reshape(_B, _H, _W)
  mins, maxs, hist = _sc_call(v3)

  alphas = pl.pallas_call(
      _mlp_body,
      out_shape=jax.ShapeDtypeStruct((_B, _ITERS), jnp.float32),
  )(hist.reshape(_NW, _NBINS), mins.reshape(_NW, _L), maxs.reshape(_NW, _L),
    mu, W1, b1, W2, b2, W3, b3, W4, b4, W5, b5)

  out = pl.pallas_call(
      _curve_body,
      grid=(_B // _CB,),
      in_specs=[
          pl.BlockSpec((_B, _ITERS), lambda b: (0, 0),
                       memory_space=pltpu.SMEM),
          pl.BlockSpec((_CB, _H, _W), lambda b: (b, 0, 0)),
      ],
      out_specs=pl.BlockSpec((_CB, _H, _W), lambda b: (b, 0, 0)),
      out_shape=jax.ShapeDtypeStruct((_B, _H, _W), jnp.float32),
  )(alphas, v3)
  return out.reshape(V_chanel.shape)


# speedup vs baseline: 3.6571x; 1.0108x over previous
import functools

import jax
import jax.numpy as jnp
from jax import lax
from jax.experimental import pallas as pl
from jax.experimental.pallas import tpu as pltpu
from jax.experimental.pallas import tpu_sc as plsc

_NBINS = 256
_MID = 64
_ITERS = 8
_NC, _NS, _L = 2, 16, 16
_NW = _NC * _NS
_B = 16
_H = 512
_W = 512
_HW = _H * _W
_HALF = _HW // 2
_CHR = 64
_NCH = (_H // 2) // _CHR
_U = 8
_UH = 16

_mesh = plsc.VectorSubcoreMesh(
    core_axis_name="c", subcore_axis_name="s",
    num_cores=_NC, num_subcores=_NS)


def _sc_body(v_hbm, mins_hbm, maxs_hbm, hist_hbm, bufs, hvals, stage, stage2,
             shared, sem0, sem1):
  c = lax.axis_index("c")
  s = lax.axis_index("s")
  wid = c * _NS + s
  b = wid // 2
  row0 = (wid % 2) * (_H // 2)
  sems = (sem0, sem1)

  def src(k):
    return v_hbm.at[b, pl.ds(row0 + k * _CHR, _CHR), :]

  mns = list(jnp.full((_L,), jnp.inf, jnp.float32) for _ in range(_U))
  mxs = list(jnp.full((_L,), -jnp.inf, jnp.float32) for _ in range(_U))
  pend = pltpu.async_copy(src(0), bufs.at[0], sems[0])
  for k in range(_NCH):
    nxt = None
    if k + 1 < _NCH:
      nxt = pltpu.async_copy(src(k + 1), bufs.at[(k + 1) % 2],
                             sems[(k + 1) % 2])
    pend.wait()

    def step(i, carry2, _k=k):
      mns2, mxs2 = carry2
      new_mns, new_mxs = list(mns2), list(mxs2)
      row = i >> 1
      col = (i & 1) * (_W // 2)
      for u in range(_W // (2 * _L)):
        x = bufs[_k % 2, row, pl.ds(col + u * _L, _L)]
        new_mns[u % _U] = jnp.minimum(new_mns[u % _U], x)
        new_mxs[u % _U] = jnp.maximum(new_mxs[u % _U], x)
      return tuple(new_mns), tuple(new_mxs)

    mns, mxs = plsc.parallel_loop(
        0, 2 * _CHR, carry=(tuple(mns), tuple(mxs)))(step)
    pend = nxt
  own_mn = functools.reduce(jnp.minimum, mns)
  own_mx = functools.reduce(jnp.maximum, mxs)
  stage[pl.ds(0, _L)] = own_mn
  stage[pl.ds(_L, _L)] = own_mx
  pltpu.sync_copy(stage.at[pl.ds(0, _L)], mins_hbm.at[pl.ds(wid * _L, _L)])
  pltpu.sync_copy(stage.at[pl.ds(_L, _L)], maxs_hbm.at[pl.ds(wid * _L, _L)])
  pltpu.sync_copy(stage, shared.at[s])
  plsc.subcore_barrier()
  pltpu.sync_copy(shared.at[s ^ 1], stage2)
  mnv = jnp.minimum(own_mn, stage2[pl.ds(0, _L)])
  mxv = jnp.maximum(own_mx, stage2[pl.ds(_L, _L)])
  mn_s = mnv[0]
  mx_s = mxv[0]
  for i in range(1, _L):
    mn_s = jnp.minimum(mn_s, mnv[i])
    mx_s = jnp.maximum(mx_s, mxv[i])
  mn = jnp.broadcast_to(mn_s, (_L,))
  mx = jnp.broadcast_to(mx_s, (_L,))
  rng = mx - mn
  safe = jnp.where(rng == 0.0, jnp.float32(1.0), rng)
  inv = jnp.float32(_NBINS) / safe

  zero = jnp.zeros((_L,), jnp.float32)
  for j in range(_NBINS // _L):
    hvals[pl.ds(j * _L, _L)] = zero

  ones = jnp.ones((_L,), jnp.float32)
  pend = pltpu.async_copy(src(0), bufs.at[0], sems[0])
  for k in range(_NCH):
    nxt = None
    if k + 1 < _NCH:
      nxt = pltpu.async_copy(src(k + 1), bufs.at[(k + 1) % 2],
                             sems[(k + 1) % 2])
    pend.wait()

    def step(i, _k=k):
      row = i >> 1
      col = (i & 1) * (_W // 2)
      for u in range(_W // (2 * _L)):
        x = bufs[_k % 2, row, pl.ds(col + u * _L, _L)]
        t = (x - mn) * inv
        idx = jnp.minimum(t, jnp.float32(_NBINS - 1)).astype(jnp.int32)
        plsc.addupdate_scatter(hvals, [idx], ones)

    plsc.parallel_loop(0, 2 * _CHR)(step)
    pend = nxt

  pltpu.sync_copy(hvals, hist_hbm.at[pl.ds(wid * _NBINS, _NBINS)])


_sc_call = pl.kernel(
    _sc_body,
    out_type=(jax.ShapeDtypeStruct((_NW * _L,), jnp.float32),
              jax.ShapeDtypeStruct((_NW * _L,), jnp.float32),
              jax.ShapeDtypeStruct((_NW * _NBINS,), jnp.float32)),
    mesh=_mesh,
    scratch_types=[pltpu.VMEM((2, _CHR, _W), jnp.float32),
                   pltpu.VMEM((_NBINS,), jnp.float32),
                   pltpu.VMEM((2 * _L,), jnp.float32),
                   pltpu.VMEM((2 * _L,), jnp.float32),
                   pltpu.VMEM_SHARED((_NS, 2 * _L), jnp.float32),
                   pltpu.SemaphoreType.DMA,
                   pltpu.SemaphoreType.DMA],
    compiler_params=pltpu.CompilerParams(needs_layout_passes=False,
                                         use_tc_tiling_on_sc=True),
)


def _lrelu(x):
  return jnp.where(x >= 0, x, 0.01 * x)


def _mlp_body(hist_ref, mins_ref, maxs_ref, mu_ref,
              w1, b1, w2, b2, w3, b3, w4, b4, w5, b5, out_ref):
  h3 = hist_ref[...].reshape(_B, 2, _NBINS)
  counts = h3[:, 0, :] + h3[:, 1, :]
  h = counts * jnp.float32(1.0 / _HW)
  m3 = mins_ref[...].reshape(_B, 2, _L)
  x3 = maxs_ref[...].reshape(_B, 2, _L)
  mn = jnp.min(jnp.minimum(m3[:, 0, :], m3[:, 1, :]), axis=1, keepdims=True)
  mx = jnp.max(jnp.maximum(x3[:, 0, :], x3[:, 1, :]), axis=1, keepdims=True)
  vec = jnp.concatenate([h, mn, mx, mu_ref[...]], axis=1)
  x = _lrelu(vec @ w1[...] + b1[...])
  x = _lrelu(x @ w2[...] + b2[...])
  x = _lrelu(jnp.concatenate([x, vec], axis=1) @ w3[...] + b3[...])
  x = _lrelu(x @ w4[...] + b4[...])
  out_ref[...] = _lrelu(x @ w5[...] + b5[...])


_CB = 4


def _curve_body(a_ref, v_ref, o_ref):
  g = pl.program_id(0)
  for j in range(_CB):
    x = v_ref[j]
    for i in range(_ITERS):
      a = a_ref[g * _CB + j, i]
      x = x * ((1.0 + a) - a * x)
    o_ref[j] = x


def kernel(V_chanel, mu, W1, b1, W2, b2, W3, b3, W4, b4, W5, b5):
  v3 = V_chanel.reshape(_B, _H, _W)
  mins, maxs, hist = _sc_call(v3)

  alphas = pl.pallas_call(
      _mlp_body,
      out_shape=jax.ShapeDtypeStruct((_B, _ITERS), jnp.float32),
  )(hist.reshape(_NW, _NBINS), mins.reshape(_NW, _L), maxs.reshape(_NW, _L),
    mu, W1, b1, W2, b2, W3, b3, W4, b4, W5, b5)

  out = pl.pallas_call(
      _curve_body,
      grid=(_B // _CB,),
      in_specs=[
          pl.BlockSpec((_B, _ITERS), lambda b: (0, 0),
                       memory_space=pltpu.SMEM),
          pl.BlockSpec((_CB, _H, _W), lambda b: (b, 0, 0)),
      ],
      out_specs=pl.BlockSpec((_CB, _H, _W), lambda b: (b, 0, 0)),
      out_shape=jax.ShapeDtypeStruct((_B, _H, _W), jnp.float32),
  )(alphas, v3)
  return out.reshape(V_chanel.shape)
